# baseline (speedup 1.0000x reference)
import functools

import jax
import jax.numpy as jnp
from jax import lax
from jax.experimental import pallas as pl
from jax.experimental.pallas import tpu as pltpu
from jax.experimental.pallas import tpu_sc as plsc

B = 16384
NUM_BINS = 10
D = 64
V = 100001
EPS = 1e-5

NC = 2
NS = 16
NW = NC * NS
BPW = B // NW

NCH = (V + 127) // 128
VPAD = NCH * 128
TSTART = (NCH - 1) * 128
TPADW = 40


def _mesh():
    return plsc.VectorSubcoreMesh(core_axis_name="c", subcore_axis_name="s",
                                  num_cores=NC, num_subcores=NS)


@functools.lru_cache(maxsize=None)
def _get_sc_gather(rows):
    @functools.partial(
        pl.kernel,
        mesh=_mesh(),
        out_type=jax.ShapeDtypeStruct((B, D), jnp.float32),
        scratch_types=[
            pltpu.VMEM((BPW,), jnp.int32),
            pltpu.VMEM((BPW, D), jnp.float32),
            pltpu.SemaphoreType.DMA,
        ],
    )
    def _sc_gather(table, idx, out, idx_v, rows_v, sem):
        wid = lax.axis_index("s") * NC + lax.axis_index("c")
        base = wid * BPW
        pltpu.sync_copy(idx.at[pl.ds(base, BPW)], idx_v)

        def fire(g, carry):
            v = idx_v[pl.ds(g * 16, 16)]
            for l in range(16):
                pltpu.async_copy(table.at[pl.ds(v[l], 1)],
                                 rows_v.at[pl.ds(g * 16 + l, 1)], sem)
            return carry

        lax.fori_loop(0, BPW // 16, fire, 0)

        def drain(j, carry):
            pltpu.make_async_copy(table.at[pl.ds(0, 1)],
                                  rows_v.at[pl.ds(j, 1)], sem).wait()
            return carry

        lax.fori_loop(0, BPW, drain, 0)
        pltpu.sync_copy(rows_v, out.at[pl.ds(base, BPW)])

    return _sc_gather


@functools.lru_cache(maxsize=None)
def _get_sc_transpose():
    @functools.partial(
        pl.kernel,
        mesh=_mesh(),
        out_type=jax.ShapeDtypeStruct((VPAD, D), jnp.float32),
        scratch_types=[
            pltpu.VMEM((D, 128), jnp.float32),
            pltpu.VMEM((D, 128), jnp.float32),
            pltpu.VMEM((128, D), jnp.float32),
            pltpu.VMEM((128, D), jnp.float32),
            pltpu.SemaphoreType.DMA,
            pltpu.SemaphoreType.DMA,
            pltpu.SemaphoreType.DMA,
            pltpu.SemaphoreType.DMA,
        ],
        compiler_params=pltpu.CompilerParams(needs_layout_passes=False),
    )
    def _sc_transpose(tableT, out, in_a, in_b, out_a, out_b, sia, sib, soa, sob):
        wid = lax.axis_index("s") * NC + lax.axis_index("c")
        lanes = lax.iota(jnp.int32, 16)

        def start_in(c, buf, sem):
            pltpu.async_copy(tableT.at[:, pl.ds(c * 128, 128)], buf, sem)

        def shuffle(in_buf, out_vm):
            dvs = [lanes + (16 * kk) for kk in range(4)]

            def rows4(j, rvc):
                for u in range(4):
                    rv = rvc + u
                    for kk in range(4):
                        v = plsc.load_gather(in_buf, [dvs[kk], rv])
                        plsc.store_scatter(out_vm, [rv, dvs[kk]], v)
                return rvc + 4

            lax.fori_loop(0, 32, rows4, jnp.zeros((16,), jnp.int32))

        def half(j2, c, in_buf, out_vm, si, so):
            @pl.when(c < NCH - 1)
            def _():
                pltpu.make_async_copy(tableT.at[:, pl.ds(c * 128, 128)],
                                      in_buf, si).wait()

                @pl.when(j2 > 0)
                def _():
                    pltpu.make_async_copy(out_vm, out.at[pl.ds(0, 128)],
                                          so).wait()

                shuffle(in_buf, out_vm)
                pltpu.async_copy(out_vm, out.at[pl.ds(c * 128, 128)], so)

        npair = (NCH - 1 + 2 * NW - 1) // (2 * NW)

        @pl.when(wid < NCH - 1)
        def _():
            start_in(wid, in_a, sia)

        @pl.when(NW + wid < NCH - 1)
        def _():
            start_in(NW + wid, in_b, sib)

        def pair(j2, carry):
            ca = (2 * j2) * NW + wid
            cb = (2 * j2 + 1) * NW + wid
            na = (2 * j2 + 2) * NW + wid
            nb = (2 * j2 + 3) * NW + wid
            half(j2, ca, in_a, out_a, sia, soa)

            @pl.when(na < NCH - 1)
            def _():
                start_in(na, in_a, sia)

            half(j2, cb, in_b, out_b, sib, sob)

            @pl.when(nb < NCH - 1)
            def _():
                start_in(nb, in_b, sib)

            return carry

        lax.fori_loop(0, npair, pair, 0)

        @pl.when(wid < NCH - 1)
        def _():
            pltpu.make_async_copy(out_a, out.at[pl.ds(0, 128)], soa).wait()

        @pl.when(NW + wid < NCH - 1)
        def _():
            pltpu.make_async_copy(out_b, out.at[pl.ds(0, 128)], sob).wait()

    return _sc_transpose


def _layernorm(t, gamma, beta):
    mu = jnp.mean(t, axis=0, keepdims=True)
    xc = t - mu
    var = jnp.mean(xc * xc, axis=0, keepdims=True)
    return xc * lax.rsqrt(var + EPS) * gamma + beta


def _tc_body(nums_ref, cats_ref, g0_ref, g1_ref, g2_ref, g3_ref, centers_ref,
             w_ref, bias_ref, tails_ref, gamma_ref, beta_ref, out_ref):
    gamma = gamma_ref[...]
    beta = beta_ref[...]
    for f in range(4):
        x = nums_ref[f:f + 1, :]
        mask = jnp.isnan(x)
        clean = jnp.where(mask, 0.0, x)
        d = -((clean - centers_ref[:, f:f + 1]) ** 2)
        d = d - jnp.max(d, axis=0, keepdims=True)
        e = jnp.exp(d)
        p = e / jnp.sum(e, axis=0, keepdims=True)
        tok = jnp.dot(w_ref[f], p, preferred_element_type=jnp.float32)
        tok = tok + bias_ref[:, f:f + 1]
        tok = jnp.where(mask, 0.0, tok)
        out_ref[f] = _layernorm(tok, gamma, beta)
    for f, g_ref in enumerate((g0_ref, g1_ref, g2_ref, g3_ref)):
        t = jnp.transpose(g_ref[...], (1, 0))
        idx = cats_ref[f:f + 1, :]
        sel = idx >= TSTART
        offs = idx - TSTART
        oh = (lax.broadcasted_iota(jnp.int32, (TPADW,) + offs.shape[1:], 0)
              == offs).astype(jnp.float32)
        fix = jnp.dot(tails_ref[f], oh, preferred_element_type=jnp.float32)
        t = jnp.where(sel, fix, t)
        out_ref[4 + f] = _layernorm(t, gamma, beta)


BN = 2048


def _tc_call(nums, cats, gs, centers, w, bias, tails, gamma, beta,
             interpret=False):
    grid = B // BN
    gspec = pl.BlockSpec((BN, D), lambda i: (i, 0))
    return pl.pallas_call(
        _tc_body,
        grid=(grid,),
        in_specs=[
            pl.BlockSpec((4, BN), lambda i: (0, i)),
            pl.BlockSpec((4, BN), lambda i: (0, i)),
            gspec, gspec, gspec, gspec,
            pl.BlockSpec((NUM_BINS, 4), lambda i: (0, 0)),
            pl.BlockSpec((4, D, NUM_BINS), lambda i: (0, 0, 0)),
            pl.BlockSpec((D, 4), lambda i: (0, 0)),
            pl.BlockSpec((4, D, TPADW), lambda i: (0, 0, 0)),
            pl.BlockSpec((D, 1), lambda i: (0, 0)),
            pl.BlockSpec((D, 1), lambda i: (0, 0)),
        ],
        out_specs=pl.BlockSpec((8, D, BN), lambda i: (0, 0, i)),
        out_shape=jax.ShapeDtypeStruct((8, D, B), jnp.float32),
        interpret=interpret,
    )(nums, cats, *gs, centers, w, bias, tails, gamma, beta)


@jax.jit
def kernel(num_0, num_1, num_2, num_3, cat_0, cat_1, cat_2, cat_3,
           centers_0, centers_1, centers_2, centers_3,
           W_0, W_1, W_2, W_3, b_0, b_1, b_2, b_3,
           E_0, E_1, E_2, E_3, gamma, beta):
    gather_v = _get_sc_gather(V)
    gather_p = _get_sc_gather(VPAD)
    transpose = _get_sc_transpose()
    gs = [gather_v(E_0, cat_0), gather_v(E_1, cat_1),
          gather_p(transpose(E_2.T), cat_2),
          gather_p(transpose(E_3.T), cat_3)]
    nums = jnp.stack([num_0, num_1, num_2, num_3], axis=0)
    cats = jnp.stack([cat_0, cat_1, cat_2, cat_3], axis=0)
    centers = jnp.stack([centers_0, centers_1, centers_2, centers_3], axis=1)
    w = jnp.stack([W_0, W_1, W_2, W_3], axis=0)
    bias = jnp.stack([b_0, b_1, b_2, b_3], axis=1)
    tails = jnp.stack(
        [jnp.pad(E.T[:, TSTART:], ((0, 0), (0, TPADW - (V - TSTART))))
         for E in (E_0, E_1, E_2, E_3)], axis=0)
    out = _tc_call(nums, cats, gs, centers, w, bias, tails,
                   gamma[:, None], beta[:, None])
    return jnp.transpose(out, (2, 0, 1))


# speedup vs baseline: 1.9606x; 1.9606x over previous
import functools

import jax
import jax.numpy as jnp
from jax import lax
from jax.experimental import pallas as pl
from jax.experimental.pallas import tpu as pltpu
from jax.experimental.pallas import tpu_sc as plsc

B = 16384
NUM_BINS = 10
D = 64
EPS = 1e-5

NC = 2
NS = 16
NW = NC * NS
BPW = B // NW


@functools.lru_cache(maxsize=None)
def _get_sc_gather():
    mesh = plsc.VectorSubcoreMesh(core_axis_name="c", subcore_axis_name="s",
                                  num_cores=NC, num_subcores=NS)

    @functools.partial(
        pl.kernel,
        mesh=mesh,
        out_type=jax.ShapeDtypeStruct((B, D), jnp.float32),
        scratch_types=[
            pltpu.VMEM((BPW,), jnp.int32),
            pltpu.VMEM((BPW, D), jnp.float32),
            pltpu.SemaphoreType.DMA,
        ],
    )
    def _sc_gather(table, idx, out, idx_v, rows_v, sem):
        wid = lax.axis_index("s") * NC + lax.axis_index("c")
        base = wid * BPW
        pltpu.sync_copy(idx.at[pl.ds(base, BPW)], idx_v)

        def fire(g, carry):
            v = idx_v[pl.ds(g * 16, 16)]
            for l in range(16):
                pltpu.async_copy(table.at[pl.ds(v[l], 1)],
                                 rows_v.at[pl.ds(g * 16 + l, 1)], sem)
            return carry

        lax.fori_loop(0, BPW // 16, fire, 0)

        def drain(j, carry):
            pltpu.make_async_copy(table.at[pl.ds(0, 1)],
                                  rows_v.at[pl.ds(j, 1)], sem).wait()
            return carry

        lax.fori_loop(0, BPW, drain, 0)
        pltpu.sync_copy(rows_v, out.at[pl.ds(base, BPW)])

    return _sc_gather


def _layernorm(t, gamma, beta):
    mu = jnp.mean(t, axis=0, keepdims=True)
    xc = t - mu
    var = jnp.mean(xc * xc, axis=0, keepdims=True)
    return xc * lax.rsqrt(var + EPS) * gamma + beta


def _tc_body1(nums_ref, g0_ref, g1_ref, centers_ref, w_ref,
              bias_ref, gamma_ref, beta_ref, out_ref):
    gamma = gamma_ref[...]
    beta = beta_ref[...]
    for f in range(4):
        x = nums_ref[f:f + 1, :]
        mask = jnp.isnan(x)
        clean = jnp.where(mask, 0.0, x)
        d = -((clean - centers_ref[:, f:f + 1]) ** 2)
        d = d - jnp.max(d, axis=0, keepdims=True)
        e = jnp.exp(d)
        p = e / jnp.sum(e, axis=0, keepdims=True)
        tok = jnp.dot(w_ref[f], p, preferred_element_type=jnp.float32)
        tok = tok + bias_ref[:, f:f + 1]
        tok = jnp.where(mask, 0.0, tok)
        out_ref[f] = _layernorm(tok, gamma, beta)
    for f, g_ref in enumerate((g0_ref, g1_ref)):
        t = jnp.transpose(g_ref[...], (1, 0))
        out_ref[4 + f] = _layernorm(t, gamma, beta)


def _tc_body2(prev_ref, g2_ref, g3_ref, gamma_ref, beta_ref, out_ref):
    del prev_ref
    gamma = gamma_ref[...]
    beta = beta_ref[...]
    for f, g_ref in enumerate((g2_ref, g3_ref)):
        t = jnp.transpose(g_ref[...], (1, 0))
        out_ref[f] = _layernorm(t, gamma, beta)


BN = 2048


def _tc_call(nums, gs, centers, w, bias, gamma, beta, interpret=False):
    grid = B // BN
    gspec = pl.BlockSpec((BN, D), lambda i: (i, 0))
    out1 = pl.pallas_call(
        _tc_body1,
        grid=(grid,),
        in_specs=[
            pl.BlockSpec((4, BN), lambda i: (0, i)),
            gspec, gspec,
            pl.BlockSpec((NUM_BINS, 4), lambda i: (0, 0)),
            pl.BlockSpec((4, D, NUM_BINS), lambda i: (0, 0, 0)),
            pl.BlockSpec((D, 4), lambda i: (0, 0)),
            pl.BlockSpec((D, 1), lambda i: (0, 0)),
            pl.BlockSpec((D, 1), lambda i: (0, 0)),
        ],
        out_specs=pl.BlockSpec((8, D, BN), lambda i: (0, 0, i)),
        out_shape=jax.ShapeDtypeStruct((8, D, B), jnp.float32),
        interpret=interpret,
    )(nums, gs[0], gs[1], centers, w, bias, gamma, beta)
    return pl.pallas_call(
        _tc_body2,
        grid=(grid,),
        in_specs=[
            pl.BlockSpec(memory_space=pl.ANY),
            gspec, gspec,
            pl.BlockSpec((D, 1), lambda i: (0, 0)),
            pl.BlockSpec((D, 1), lambda i: (0, 0)),
        ],
        out_specs=pl.BlockSpec((2, D, BN), lambda i: (3, 0, i)),
        out_shape=jax.ShapeDtypeStruct((8, D, B), jnp.float32),
        input_output_aliases={0: 0},
        interpret=interpret,
    )(out1, gs[2], gs[3], gamma, beta)


@jax.jit
def kernel(num_0, num_1, num_2, num_3, cat_0, cat_1, cat_2, cat_3,
           centers_0, centers_1, centers_2, centers_3,
           W_0, W_1, W_2, W_3, b_0, b_1, b_2, b_3,
           E_0, E_1, E_2, E_3, gamma, beta):
    sc_gather = _get_sc_gather()
    gs = [sc_gather(E, c) for E, c in
          ((E_0, cat_0), (E_1, cat_1), (E_2, cat_2), (E_3, cat_3))]
    nums = jnp.stack([num_0, num_1, num_2, num_3], axis=0)
    centers = jnp.stack([centers_0, centers_1, centers_2, centers_3], axis=1)
    w = jnp.stack([W_0, W_1, W_2, W_3], axis=0)
    bias = jnp.stack([b_0, b_1, b_2, b_3], axis=1)
    out = _tc_call(nums, gs, centers, w, bias, gamma[:, None], beta[:, None])
    return jnp.transpose(out, (2, 0, 1))


# speedup vs baseline: 1.9839x; 1.0119x over previous
import functools

import jax
import jax.numpy as jnp
from jax import lax
from jax.experimental import pallas as pl
from jax.experimental.pallas import tpu as pltpu
from jax.experimental.pallas import tpu_sc as plsc

B = 16384
NUM_BINS = 10
D = 64
EPS = 1e-5

NC = 2
NS = 16
NW = NC * NS
BPW = B // NW


@functools.lru_cache(maxsize=None)
def _get_sc_gather():
    mesh = plsc.VectorSubcoreMesh(core_axis_name="c", subcore_axis_name="s",
                                  num_cores=NC, num_subcores=NS)

    @functools.partial(
        pl.kernel,
        mesh=mesh,
        out_type=jax.ShapeDtypeStruct((B, D), jnp.float32),
        scratch_types=[
            pltpu.VMEM((BPW,), jnp.int32),
            pltpu.VMEM((BPW, D), jnp.float32),
            pltpu.SemaphoreType.DMA,
        ],
    )
    def _sc_gather(table, idx, out, idx_v, rows_v, sem):
        wid = lax.axis_index("s") * NC + lax.axis_index("c")
        base = wid * BPW
        pltpu.sync_copy(idx.at[pl.ds(base, BPW)], idx_v)

        def fire(g, carry):
            v = idx_v[pl.ds(g * 16, 16)]
            for l in range(16):
                pltpu.async_copy(table.at[pl.ds(v[l], 1)],
                                 rows_v.at[pl.ds(g * 16 + l, 1)], sem)
            return carry

        lax.fori_loop(0, BPW // 16, fire, 0)

        def drain(j, carry):
            pltpu.make_async_copy(table.at[pl.ds(0, 1)],
                                  rows_v.at[pl.ds(j, 1)], sem).wait()
            return carry

        lax.fori_loop(0, BPW, drain, 0)
        pltpu.sync_copy(rows_v, out.at[pl.ds(base, BPW)])

    return _sc_gather


def _layernorm(t, gamma, beta):
    mu = jnp.mean(t, axis=0, keepdims=True)
    xc = t - mu
    var = jnp.mean(xc * xc, axis=0, keepdims=True)
    return xc * lax.rsqrt(var + EPS) * gamma + beta


def _tc_body(nums_ref, g0_ref, g1_ref, g2_ref, g3_ref, centers_ref, w_ref,
             bias_ref, gamma_ref, beta_ref, out_ref):
    gamma = gamma_ref[...]
    beta = beta_ref[...]
    for f in range(4):
        x = nums_ref[f:f + 1, :]
        mask = jnp.isnan(x)
        clean = jnp.where(mask, 0.0, x)
        d = -((clean - centers_ref[:, f:f + 1]) ** 2)
        d = d - jnp.max(d, axis=0, keepdims=True)
        e = jnp.exp(d)
        p = e / jnp.sum(e, axis=0, keepdims=True)
        tok = jnp.dot(w_ref[f], p, preferred_element_type=jnp.float32)
        tok = tok + bias_ref[:, f:f + 1]
        tok = jnp.where(mask, 0.0, tok)
        out_ref[f] = _layernorm(tok, gamma, beta)
    for f, g_ref in enumerate((g0_ref, g1_ref, g2_ref, g3_ref)):
        t = jnp.transpose(g_ref[...], (1, 0))
        out_ref[4 + f] = _layernorm(t, gamma, beta)


BN = 2048


def _tc_call(nums, gs, centers, w, bias, gamma, beta, interpret=False):
    grid = B // BN
    gspec = pl.BlockSpec((BN, D), lambda i: (i, 0))
    return pl.pallas_call(
        _tc_body,
        grid=(grid,),
        in_specs=[
            pl.BlockSpec((4, BN), lambda i: (0, i)),
            gspec, gspec, gspec, gspec,
            pl.BlockSpec((NUM_BINS, 4), lambda i: (0, 0)),
            pl.BlockSpec((4, D, NUM_BINS), lambda i: (0, 0, 0)),
            pl.BlockSpec((D, 4), lambda i: (0, 0)),
            pl.BlockSpec((D, 1), lambda i: (0, 0)),
            pl.BlockSpec((D, 1), lambda i: (0, 0)),
        ],
        out_specs=pl.BlockSpec((8, D, BN), lambda i: (0, 0, i)),
        out_shape=jax.ShapeDtypeStruct((8, D, B), jnp.float32),
        interpret=interpret,
    )(nums, *gs, centers, w, bias, gamma, beta)


@jax.jit
def kernel(num_0, num_1, num_2, num_3, cat_0, cat_1, cat_2, cat_3,
           centers_0, centers_1, centers_2, centers_3,
           W_0, W_1, W_2, W_3, b_0, b_1, b_2, b_3,
           E_0, E_1, E_2, E_3, gamma, beta):
    sc_gather = _get_sc_gather()
    gs = [sc_gather(E, c) for E, c in
          ((E_0, cat_0), (E_1, cat_1), (E_2, cat_2), (E_3, cat_3))]
    nums = jnp.stack([num_0, num_1, num_2, num_3], axis=0)
    centers = jnp.stack([centers_0, centers_1, centers_2, centers_3], axis=1)
    w = jnp.stack([W_0, W_1, W_2, W_3], axis=0)
    bias = jnp.stack([b_0, b_1, b_2, b_3], axis=1)
    out = _tc_call(nums, gs, centers, w, bias, gamma[:, None], beta[:, None])
    return jnp.transpose(out, (2, 0, 1))
